# Initial kernel scaffold; baseline (speedup 1.0000x reference)
import jax
import jax.numpy as jnp
from jax.experimental import pallas as pl


def kernel(m, s, o, N_p, co_A, co_B, co_C, co_D, sigma_u, sigma_v):
    raise NotImplementedError("write your pallas kernel here")


# speedup vs baseline: 1.5963x; 1.5963x over previous
import jax
import jax.numpy as jnp
from jax.experimental import pallas as pl
from jax.experimental.pallas import tpu as pltpu


def _rowsum(x):
    acc = x[:, 0:128]
    for c in range(1, 8):
        acc = acc + x[:, 128 * c:128 * (c + 1)]
    a2 = acc[:, 0:8]
    for j in range(1, 16):
        a2 = a2 + acc[:, 8 * j:8 * (j + 1)]
    a2 = a2[:, 0:4] + a2[:, 4:8]
    a2 = a2[:, 0:2] + a2[:, 2:4]
    return a2[:, 0:1] + a2[:, 1:2]


def _filter_body(params_ref, s0_ref, o_ref, noise_ref, u_ref, est_ref,
                 s_scr, w_scr, scan_scr, sres_scr, rec_scr):
    t = pl.program_id(0)
    R, Np = s_scr.shape
    nb = Np // 128

    @pl.when(t == 0)
    def _init():
        s_scr[...] = s0_ref[...]
        w_scr[...] = jnp.full((R, Np), 1.0 / Np, dtype=jnp.float32)

    A = params_ref[0]
    B = params_ref[1]
    C = params_ref[2]
    D = params_ref[3]
    su = params_ref[4]
    sv = params_ref[5]

    s = s_scr[...]
    w = w_scr[...]
    noise_t = noise_ref[0]
    o_t = o_ref[0]

    s = A * s + B + noise_t * su
    pred = C * jnp.sqrt(jnp.abs(s) + 1e-8) + D
    lik = jnp.exp(-0.5 * ((o_t - pred) / sv) ** 2) / (sv * jnp.sqrt(2.0 * jnp.pi))
    w = w * lik + 1e-30
    total = _rowsum(w)
    rec_scr[:, 0:1] = pl.reciprocal(total, approx=True)
    w = w * rec_scr[:, 0:1]
    ess_sum = _rowsum(w ** 2)
    trig = (1.0 / ess_sum) < jnp.float32(Np)

    scan_scr[...] = jnp.transpose(w)
    for i in range(1, 128):
        for c in range(nb):
            r0 = 128 * c + i
            scan_scr[r0:r0 + 1, :] = (scan_scr[r0:r0 + 1, :]
                                      + scan_scr[r0 - 1:r0, :])
    off = scan_scr[127:128, :]
    for c in range(1, nb):
        scan_scr[128 * c:128 * (c + 1), :] = (
            scan_scr[128 * c:128 * (c + 1), :] + off)
        off = scan_scr[128 * c + 127:128 * c + 128, :]

    s_scr[...] = s
    cumT = scan_scr[...]

    def row_body(r, _):
        cum_col = pltpu.roll(cumT, R - r, axis=1)[:, 0:1]
        u_row = u_ref[0, pl.ds(r, 1), :]
        s_row = s_scr[pl.ds(r, 1), :]
        S = (cum_col >= u_row).astype(jnp.float32)
        S = jnp.concatenate(
            [S[:Np - 1, :], jnp.ones((1, Np), jnp.float32)], axis=0)
        onehot = S - jnp.concatenate(
            [jnp.zeros((1, Np), jnp.float32), S[:Np - 1, :]], axis=0)
        sres = jax.lax.dot_general(
            s_row, onehot, (((1,), (0,)), ((), ())),
            preferred_element_type=jnp.float32,
            precision=jax.lax.Precision.HIGHEST)
        sres_scr[pl.ds(r, 1), :] = sres
        return 0

    jax.lax.fori_loop(0, R, row_body, 0)

    s_new = jnp.where(trig, sres_scr[...], s_scr[...])
    w_new = jnp.where(trig, jnp.float32(1.0 / Np), w)
    est_ref[0] = _rowsum(w_new * s_new)
    s_scr[...] = s_new
    w_scr[...] = w_new


def kernel(m, s, o, N_p, co_A, co_B, co_C, co_D, sigma_u, sigma_v):
    batch = o.shape[0]
    T = o.shape[2]
    Np = s.shape[1]

    key = jax.random.key(42)
    s0 = jax.random.uniform(
        jax.random.fold_in(key, 0), (batch, Np), minval=-0.5, maxval=0.5)
    noise = jnp.stack(
        [jax.random.normal(jax.random.fold_in(key, 3 * t), (batch, Np))
         for t in range(1, T)], axis=0)
    u = jnp.stack(
        [jax.random.uniform(jax.random.fold_in(key, 3 * t + 1), (batch, Np))
         for t in range(1, T)], axis=0)
    o_steps = jnp.transpose(o[:, 0, 1:].astype(jnp.float32))[:, :, None]
    params = jnp.stack(
        [co_A[0], co_B[0], co_C[0], co_D[0], sigma_u[0], sigma_v[0]]
    ).astype(jnp.float32)

    est = pl.pallas_call(
        _filter_body,
        grid=(T - 1,),
        in_specs=[
            pl.BlockSpec(memory_space=pltpu.SMEM),
            pl.BlockSpec((batch, Np), lambda t: (0, 0)),
            pl.BlockSpec((1, batch, 1), lambda t: (t, 0, 0)),
            pl.BlockSpec((1, batch, Np), lambda t: (t, 0, 0)),
            pl.BlockSpec((1, batch, Np), lambda t: (t, 0, 0)),
        ],
        out_specs=pl.BlockSpec((1, batch, 1), lambda t: (t, 0, 0)),
        out_shape=jax.ShapeDtypeStruct((T - 1, batch, 1), jnp.float32),
        scratch_shapes=[
            pltpu.VMEM((batch, Np), jnp.float32),
            pltpu.VMEM((batch, Np), jnp.float32),
            pltpu.VMEM((Np, batch), jnp.float32),
            pltpu.VMEM((batch, Np), jnp.float32),
            pltpu.VMEM((batch, 128), jnp.float32),
        ],
    )(params, s0, o_steps, noise, u)

    w0 = jnp.full((batch, Np), 1.0 / Np, dtype=jnp.float32)
    est0 = (w0 * s0).sum(axis=1)
    ests = jnp.concatenate([est0[:, None], est[:, :, 0].T], axis=1)
    return ests[:, None, :]
